# Initial kernel scaffold; baseline (speedup 1.0000x reference)
import jax
import jax.numpy as jnp
from jax.experimental import pallas as pl


def kernel(x, W, b):
    raise NotImplementedError("write your pallas kernel here")


# speedup vs baseline: 1.0244x; 1.0244x over previous
import jax
import jax.numpy as jnp
from jax.experimental import pallas as pl

D_MODEL = 4096
N_HEADS = 64
TOP_K = 8
BLK = 512


def _router_body(x_ref, w_ref, b_ref, gates_ref, idx_ref):
    x = x_ref[...]
    w = w_ref[...]
    logits = jax.lax.dot_general(
        x, w, (((1,), (1,)), ((), ())),
        preferred_element_type=jnp.float32,
        precision=jax.lax.Precision.DEFAULT,
    )
    logits = logits + b_ref[...]

    iota = jax.lax.broadcasted_iota(jnp.int32, logits.shape, 1)
    cur = logits
    vals = []
    idxs = []
    for _ in range(TOP_K):
        m = jnp.max(cur, axis=1, keepdims=True)
        am = jnp.min(jnp.where(cur == m, iota, N_HEADS), axis=1, keepdims=True)
        vals.append(m)
        idxs.append(am)
        cur = jnp.where(iota == am, -jnp.inf, cur)
    topv = jnp.concatenate(vals, axis=1)
    topi = jnp.concatenate(idxs, axis=1)

    e = jnp.exp(topv - topv[:, :1])
    gates_ref[...] = e / jnp.sum(e, axis=1, keepdims=True)
    idx_ref[...] = topi


def kernel(x, W, b):
    B, T, D = x.shape
    n_tok = B * T
    x2 = x.reshape(n_tok, D)
    b2 = b.reshape(1, N_HEADS)
    grid = (n_tok // BLK,)
    gates, idx = pl.pallas_call(
        _router_body,
        grid=grid,
        in_specs=[
            pl.BlockSpec((BLK, D), lambda i: (i, 0)),
            pl.BlockSpec((N_HEADS, D), lambda i: (0, 0)),
            pl.BlockSpec((1, N_HEADS), lambda i: (0, 0)),
        ],
        out_specs=[
            pl.BlockSpec((BLK, TOP_K), lambda i: (i, 0)),
            pl.BlockSpec((BLK, TOP_K), lambda i: (i, 0)),
        ],
        out_shape=[
            jax.ShapeDtypeStruct((n_tok, TOP_K), jnp.float32),
            jax.ShapeDtypeStruct((n_tok, TOP_K), jnp.int32),
        ],
    )(x2, W, b2)
    return gates.reshape(B, T, TOP_K), idx.reshape(B, T, TOP_K)


# speedup vs baseline: 1.1941x; 1.1657x over previous
import jax
import jax.numpy as jnp
from jax.experimental import pallas as pl

D_MODEL = 4096
N_HEADS = 64
TOP_K = 8
BLK = 512


def _router_body(x_ref, w_ref, b_ref, gates_ref, idx_ref):
    x = x_ref[...]
    w = w_ref[...]
    logits = jax.lax.dot_general(
        x, w, (((1,), (1,)), ((), ())),
        preferred_element_type=jnp.float32,
        precision=jax.lax.Precision.DEFAULT,
    )
    logits = logits + b_ref[...]

    iota_f = jax.lax.broadcasted_iota(jnp.int32, logits.shape, 1).astype(jnp.float32)
    cur = logits
    vals = []
    idxs = []
    for _ in range(TOP_K):
        m = jnp.max(cur, axis=1, keepdims=True)
        eq = cur == m
        am = jnp.min(jnp.where(eq, iota_f, 64.0), axis=1, keepdims=True)
        vals.append(m)
        idxs.append(am)
        cur = jnp.where(eq, -jnp.inf, cur)
    topv = jnp.concatenate(vals, axis=1)
    topi = jnp.concatenate(idxs, axis=1)

    e = jnp.exp(topv - topv[:, :1])
    gates_ref[...] = e / jnp.sum(e, axis=1, keepdims=True)
    idx_ref[...] = topi.astype(jnp.int32)


def kernel(x, W, b):
    B, T, D = x.shape
    n_tok = B * T
    x2 = x.reshape(n_tok, D)
    b2 = b.reshape(1, N_HEADS)
    grid = (n_tok // BLK,)
    gates, idx = pl.pallas_call(
        _router_body,
        grid=grid,
        in_specs=[
            pl.BlockSpec((BLK, D), lambda i: (i, 0)),
            pl.BlockSpec((N_HEADS, D), lambda i: (0, 0)),
            pl.BlockSpec((1, N_HEADS), lambda i: (0, 0)),
        ],
        out_specs=[
            pl.BlockSpec((BLK, TOP_K), lambda i: (i, 0)),
            pl.BlockSpec((BLK, TOP_K), lambda i: (i, 0)),
        ],
        out_shape=[
            jax.ShapeDtypeStruct((n_tok, TOP_K), jnp.float32),
            jax.ShapeDtypeStruct((n_tok, TOP_K), jnp.int32),
        ],
    )(x2, W, b2)
    return gates.reshape(B, T, TOP_K), idx.reshape(B, T, TOP_K)


# speedup vs baseline: 1.2681x; 1.0620x over previous
import jax
import jax.numpy as jnp
from jax.experimental import pallas as pl

D_MODEL = 4096
N_HEADS = 64
TOP_K = 8
BLK = 1024


def _router_body(x_ref, w_ref, b_ref, gates_ref, idx_ref):
    x = x_ref[...]
    w = w_ref[...]
    logits = jax.lax.dot_general(
        x, w, (((1,), (1,)), ((), ())),
        preferred_element_type=jnp.float32,
        precision=jax.lax.Precision.DEFAULT,
    )
    logits = logits + b_ref[...]

    iota_f = jax.lax.broadcasted_iota(jnp.int32, logits.shape, 1).astype(jnp.float32)
    cur = logits
    vals = []
    idxs = []
    for _ in range(TOP_K):
        m = jnp.max(cur, axis=1, keepdims=True)
        eq = cur == m
        am = jnp.min(jnp.where(eq, iota_f, 64.0), axis=1, keepdims=True)
        vals.append(m)
        idxs.append(am)
        cur = jnp.where(eq, -jnp.inf, cur)
    topv = jnp.concatenate(vals, axis=1)
    topi = jnp.concatenate(idxs, axis=1)

    e = jnp.exp(topv - topv[:, :1])
    gates_ref[...] = e / jnp.sum(e, axis=1, keepdims=True)
    idx_ref[...] = topi.astype(jnp.int32)


def kernel(x, W, b):
    B, T, D = x.shape
    n_tok = B * T
    x2 = x.reshape(n_tok, D)
    b2 = b.reshape(1, N_HEADS)
    grid = (n_tok // BLK,)
    gates, idx = pl.pallas_call(
        _router_body,
        grid=grid,
        in_specs=[
            pl.BlockSpec((BLK, D), lambda i: (i, 0)),
            pl.BlockSpec((N_HEADS, D), lambda i: (0, 0)),
            pl.BlockSpec((1, N_HEADS), lambda i: (0, 0)),
        ],
        out_specs=[
            pl.BlockSpec((BLK, TOP_K), lambda i: (i, 0)),
            pl.BlockSpec((BLK, TOP_K), lambda i: (i, 0)),
        ],
        out_shape=[
            jax.ShapeDtypeStruct((n_tok, TOP_K), jnp.float32),
            jax.ShapeDtypeStruct((n_tok, TOP_K), jnp.int32),
        ],
    )(x2, W, b2)
    return gates.reshape(B, T, TOP_K), idx.reshape(B, T, TOP_K)


# speedup vs baseline: 1.8112x; 1.4283x over previous
import jax
import jax.numpy as jnp
from jax.experimental import pallas as pl

D_MODEL = 4096
N_HEADS = 64
TOP_K = 8
BLK = 1024


def _router_body(x_ref, w_ref, b_ref, gates_ref, idx_ref):
    x = x_ref[...]
    w = w_ref[...]
    logits = jax.lax.dot_general(
        w, x, (((1,), (1,)), ((), ())),
        preferred_element_type=jnp.float32,
        precision=jax.lax.Precision.DEFAULT,
    )
    logits = logits + b_ref[...]

    iota_f = jax.lax.broadcasted_iota(jnp.int32, logits.shape, 0).astype(jnp.float32)
    cur = logits
    vals = []
    idxs = []
    for k in range(TOP_K):
        m = jnp.max(cur, axis=0, keepdims=True)
        eq = cur == m
        am = jnp.min(jnp.where(eq, iota_f, 64.0), axis=0, keepdims=True)
        vals.append(m)
        idxs.append(am)
        if k + 1 < TOP_K:
            cur = jnp.where(eq, -jnp.inf, cur)
    topv = jnp.concatenate(vals, axis=0)
    topi = jnp.concatenate(idxs, axis=0)

    e = jnp.exp(topv - topv[:1])
    gates_ref[...] = e / jnp.sum(e, axis=0, keepdims=True)
    idx_ref[...] = topi.astype(jnp.int32)


def kernel(x, W, b):
    B, T, D = x.shape
    n_tok = B * T
    x2 = x.reshape(n_tok, D)
    b2 = b.reshape(N_HEADS, 1)
    grid = (n_tok // BLK,)
    gates_t, idx_t = pl.pallas_call(
        _router_body,
        grid=grid,
        in_specs=[
            pl.BlockSpec((BLK, D), lambda i: (i, 0)),
            pl.BlockSpec((N_HEADS, D), lambda i: (0, 0)),
            pl.BlockSpec((N_HEADS, 1), lambda i: (0, 0)),
        ],
        out_specs=[
            pl.BlockSpec((TOP_K, BLK), lambda i: (0, i)),
            pl.BlockSpec((TOP_K, BLK), lambda i: (0, i)),
        ],
        out_shape=[
            jax.ShapeDtypeStruct((TOP_K, n_tok), jnp.float32),
            jax.ShapeDtypeStruct((TOP_K, n_tok), jnp.int32),
        ],
    )(x2, W, b2)
    gates = gates_t.T.reshape(B, T, TOP_K)
    idx = idx_t.T.reshape(B, T, TOP_K)
    return gates, idx
